# baseline (speedup 1.0000x reference)
import jax
import jax.numpy as jnp
from jax.experimental import pallas as pl


def _swap_kernel(x_ref, y_ref):
    y_ref[...] = x_ref[...].reshape(y_ref.shape)


def _major_swap(x, h0, nh):
    c, fh, bs, fw = x.shape
    return pl.pallas_call(
        _swap_kernel,
        grid=(nh,),
        in_specs=[pl.BlockSpec((c, 1, bs, fw), lambda h: (0, h0 + h, 0, 0))],
        out_specs=pl.BlockSpec((1, c, bs, fw), lambda h: (h, 0, 0, 0)),
        out_shape=jax.ShapeDtypeStruct((nh, c, bs, fw), x.dtype),
    )(x)


_H_CHUNKS = ((0, 10), (10, 10), (20, 10), (30, 7))


def kernel(preds, regs):
    bs, c2, fh, fw = preds.shape
    c4 = regs.shape[1]
    rt = jnp.transpose(regs, (1, 2, 0, 3))
    pt = jnp.transpose(preds, (1, 2, 0, 3))
    ro_parts = []
    for h0, nh in _H_CHUNKS:
        qr = _major_swap(rt, h0, nh)
        ro_parts.append(
            jnp.transpose(qr, (2, 0, 3, 1)).reshape(bs, nh, fw, c4 // 4, 4))
    qo = _major_swap(pt, 0, fh)
    po = jnp.transpose(qo, (2, 0, 3, 1))
    ro = jnp.concatenate(ro_parts, axis=1)
    return po, ro


# speedup vs baseline: 1.4591x; 1.4591x over previous
import jax
import jax.numpy as jnp
from jax.experimental import pallas as pl


def _swap_kernel(x_ref, y_ref):
    y_ref[...] = x_ref[...].reshape(y_ref.shape)


def _major_swap(x):
    c, fh, bs, fw = x.shape
    return pl.pallas_call(
        _swap_kernel,
        grid=(fh,),
        in_specs=[pl.BlockSpec((c, 1, bs, fw), lambda h: (0, h, 0, 0))],
        out_specs=pl.BlockSpec((1, c, bs, fw), lambda h: (h, 0, 0, 0)),
        out_shape=jax.ShapeDtypeStruct((fh, c, bs, fw), x.dtype),
    )(x)


def kernel(preds, regs):
    bs, c2, fh, fw = preds.shape
    c4 = regs.shape[1]
    rt = jnp.transpose(regs, (1, 2, 0, 3))
    pt = jnp.transpose(preds, (1, 2, 0, 3))
    qr = _major_swap(rt)
    qo = _major_swap(pt)
    po = jnp.transpose(qo, (2, 0, 3, 1))
    ro = jnp.transpose(qr, (2, 0, 3, 1)).reshape(bs, fh, fw, c4 // 4, 4)
    return po, ro
